# Initial kernel scaffold; baseline (speedup 1.0000x reference)
import jax
import jax.numpy as jnp
from jax.experimental import pallas as pl


def kernel(center_ids, embedding, bias):
    raise NotImplementedError("write your pallas kernel here")


# speedup vs baseline: 1.2275x; 1.2275x over previous
import functools

import jax
import jax.numpy as jnp
import numpy as np
from jax import lax
from jax.experimental import pallas as pl
from jax.experimental.pallas import tpu as pltpu
from jax.experimental.pallas import tpu_sc as plsc

N_NODES = 100000
EMBED_D = 64
BATCH = 1024
TILE_N = 1000
N_TILES = N_NODES // TILE_N

_U32_9 = np.uint32(9)
_EXP_ONE = np.uint32(0x3F800000)
_MINVAL = np.float32(1e-10)
_SCALE = np.float32(np.float32(1.0) - np.float32(1e-10))


def _threefry_bits(x1):
    ks = (np.uint32(0), np.uint32(1), np.uint32(0x1BD11BDB))
    rotations = ((13, 15, 26, 6), (17, 29, 16, 24))
    x0 = jnp.zeros_like(x1)
    x1 = x1 + ks[1]
    for i in range(5):
        for r in rotations[i % 2]:
            x0 = x0 + x1
            x1 = lax.shift_left(x1, np.uint32(r)) | lax.shift_right_logical(
                x1, np.uint32(32 - r))
            x1 = x1 ^ x0
        x0 = x0 + ks[(i + 1) % 3]
        x1 = x1 + ks[(i + 2) % 3] + np.uint32(i + 1)
    return x0 ^ x1


def _fused_body(ce_ref, emb_ref, bias_ref, samp_ref, lp_ref,
                bv_ref, bi_ref, bs_ref, m_ref, s_ref):
    j = pl.program_id(0)

    @pl.when(j == 0)
    def _init():
        bv_ref[...] = jnp.full((BATCH, 1), -jnp.inf, jnp.float32)
        bi_ref[...] = jnp.zeros((BATCH, 1), jnp.int32)
        bs_ref[...] = jnp.zeros((BATCH, 1), jnp.float32)
        m_ref[...] = jnp.full((BATCH, 1), -jnp.inf, jnp.float32)
        s_ref[...] = jnp.zeros((BATCH, 1), jnp.float32)

    scores = lax.dot_general(
        ce_ref[...], emb_ref[...],
        dimension_numbers=(((1,), (1,)), ((), ())),
        preferred_element_type=jnp.float32)
    scores = scores + jnp.reshape(bias_ref[...], (1, TILE_N))

    row = lax.broadcasted_iota(jnp.int32, (BATCH, TILE_N), 0)
    col = lax.broadcasted_iota(jnp.int32, (BATCH, TILE_N), 1) + j * TILE_N
    flat = (row * N_NODES + col).astype(jnp.uint32)
    bits = _threefry_bits(flat)
    fbits = lax.shift_right_logical(bits, _U32_9) | _EXP_ONE
    u = lax.bitcast_convert_type(fbits, jnp.float32) - np.float32(1.0)
    u = jnp.maximum(_MINVAL, u * _SCALE + _MINVAL)
    t = scores - jnp.log(-jnp.log(u))

    tmax = jnp.max(t, axis=-1, keepdims=True)
    lane = lax.broadcasted_iota(jnp.int32, (BATCH, TILE_N), 1)
    larg = jnp.min(jnp.where(t == tmax, lane, TILE_N), axis=-1, keepdims=True)
    sel = jnp.sum(jnp.where(lane == larg, scores, 0.0), axis=-1, keepdims=True)

    upd = tmax > bv_ref[...]
    bv_ref[...] = jnp.where(upd, tmax, bv_ref[...])
    bi_ref[...] = jnp.where(upd, larg + j * TILE_N, bi_ref[...])
    bs_ref[...] = jnp.where(upd, sel, bs_ref[...])

    smax = jnp.max(scores, axis=-1, keepdims=True)
    m_old = m_ref[...]
    m_new = jnp.maximum(m_old, smax)
    ssum = jnp.sum(jnp.exp(scores - m_new), axis=-1, keepdims=True)
    s_ref[...] = s_ref[...] * jnp.exp(m_old - m_new) + ssum
    m_ref[...] = m_new

    @pl.when(j == N_TILES - 1)
    def _finish():
        samp_ref[...] = bi_ref[...]
        lp_ref[...] = bs_ref[...] - (m_ref[...] + jnp.log(s_ref[...]))


def _fused_tc(center_emb, embedding, bias):
    bias3d = bias.reshape(N_TILES, 1, TILE_N)
    samples2d, lp2d = pl.pallas_call(
        _fused_body,
        grid=(N_TILES,),
        in_specs=[
            pl.BlockSpec((BATCH, EMBED_D), lambda j: (0, 0)),
            pl.BlockSpec((TILE_N, EMBED_D), lambda j: (j, 0)),
            pl.BlockSpec((1, 1, TILE_N), lambda j: (j, 0, 0)),
        ],
        out_specs=[
            pl.BlockSpec((BATCH, 1), lambda j: (0, 0)),
            pl.BlockSpec((BATCH, 1), lambda j: (0, 0)),
        ],
        out_shape=[
            jax.ShapeDtypeStruct((BATCH, 1), jnp.int32),
            jax.ShapeDtypeStruct((BATCH, 1), jnp.float32),
        ],
        scratch_shapes=[
            pltpu.VMEM((BATCH, 1), jnp.float32),
            pltpu.VMEM((BATCH, 1), jnp.int32),
            pltpu.VMEM((BATCH, 1), jnp.float32),
            pltpu.VMEM((BATCH, 1), jnp.float32),
            pltpu.VMEM((BATCH, 1), jnp.float32),
        ],
        compiler_params=pltpu.CompilerParams(
            dimension_semantics=("arbitrary",)),
    )(center_emb, embedding, bias3d)
    return samples2d[:, 0], lp2d[:, 0]


def _gather_sc(center_ids, embedding):
    info = plsc.get_sparse_core_info()
    nc = info.num_cores
    b_per_c = BATCH // nc
    mesh = plsc.ScalarSubcoreMesh(axis_name="c", num_cores=nc)

    @functools.partial(
        pl.kernel, mesh=mesh,
        out_type=jax.ShapeDtypeStruct((BATCH, EMBED_D), jnp.float32),
        scratch_types=[
            pltpu.SMEM((b_per_c,), jnp.int32),
            pltpu.SemaphoreType.DMA,
            pltpu.SemaphoreType.DMA,
        ],
    )
    def gather(table_hbm, idx_hbm, out_hbm, idx_s, isem, sem):
        base = lax.axis_index("c") * b_per_c
        pltpu.async_copy(idx_hbm.at[pl.ds(base, b_per_c)], idx_s, isem).wait()

        def body(i, _):
            pltpu.async_copy(table_hbm.at[pl.ds(idx_s[i], 1), :],
                             out_hbm.at[pl.ds(base + i, 1), :], sem)
            return ()

        lax.fori_loop(0, b_per_c, body, ())
        pltpu.make_async_copy(
            table_hbm.at[pl.ds(0, b_per_c), :],
            out_hbm.at[pl.ds(base, b_per_c), :], sem).wait()

    return gather(embedding, center_ids)


def kernel(center_ids, embedding, bias):
    center_emb = _gather_sc(center_ids.astype(jnp.int32), embedding)
    return _fused_tc(center_emb, embedding, bias)
